# Initial kernel scaffold; baseline (speedup 1.0000x reference)
import jax
import jax.numpy as jnp
from jax.experimental import pallas as pl


def kernel(x, params):
    raise NotImplementedError("write your pallas kernel here")


# speedup vs baseline: 10.7687x; 10.7687x over previous
import functools

import jax
import jax.numpy as jnp
from jax import lax
from jax.experimental import pallas as pl
from jax.experimental.pallas import tpu as pltpu
from jax.experimental.pallas import tpu_sc as plsc

KNN = 20
BNEPS = 1e-5
TN = 256


def _lrelu(x):
    return jnp.where(x >= 0, x, x * jnp.float32(0.2))


def _mm(a, w, bias=None, addf=None, addrow=None, act=False, main_out=True,
        want_max=False, want_mean=False):
    B, N, C = a.shape
    O = w.shape[0]
    nt = N // TN

    in_specs = [pl.BlockSpec((1, TN, C), lambda b, n: (b, n, 0)),
                pl.BlockSpec((O, C), lambda b, n: (0, 0))]
    args = [a, w]
    if bias is not None:
        in_specs.append(pl.BlockSpec((1, O), lambda b, n: (0, 0)))
        args.append(bias.reshape(1, O))
    if addf is not None:
        in_specs.append(pl.BlockSpec((1, TN, O), lambda b, n: (b, n, 0)))
        args.append(addf)
    if addrow is not None:
        in_specs.append(pl.BlockSpec((1, 1, O), lambda b, n: (b, 0, 0)))
        args.append(addrow.reshape(B, 1, O))

    out_shape = []
    out_specs = []
    if main_out:
        out_shape.append(jax.ShapeDtypeStruct((B, N, O), jnp.float32))
        out_specs.append(pl.BlockSpec((1, TN, O), lambda b, n: (b, n, 0)))
    if want_max:
        out_shape.append(jax.ShapeDtypeStruct((B, 1, O), jnp.float32))
        out_specs.append(pl.BlockSpec((1, 1, O), lambda b, n: (b, 0, 0)))
    if want_mean:
        out_shape.append(jax.ShapeDtypeStruct((B, 1, O), jnp.float32))
        out_specs.append(pl.BlockSpec((1, 1, O), lambda b, n: (b, 0, 0)))

    def body(*refs):
        refs = list(refs)
        a_ref = refs.pop(0)
        w_ref = refs.pop(0)
        b_ref = refs.pop(0) if bias is not None else None
        af_ref = refs.pop(0) if addf is not None else None
        ar_ref = refs.pop(0) if addrow is not None else None
        o_ref = refs.pop(0) if main_out else None
        mx_ref = refs.pop(0) if want_max else None
        mn_ref = refs.pop(0) if want_mean else None
        n = pl.program_id(1)
        r = lax.dot_general(a_ref[0], w_ref[...], (((1,), (1,)), ((), ())),
                            preferred_element_type=jnp.float32)
        if b_ref is not None:
            r = r + b_ref[...]
        if af_ref is not None:
            r = r + af_ref[0]
        if ar_ref is not None:
            r = r + ar_ref[0]
        if act:
            r = _lrelu(r)
        if o_ref is not None:
            o_ref[0] = r
        if mx_ref is not None:
            @pl.when(n == 0)
            def _():
                mx_ref[0] = jnp.full((1, O), -jnp.inf, jnp.float32)
            mx_ref[0] = jnp.maximum(mx_ref[0], jnp.max(r, axis=0, keepdims=True))
        if mn_ref is not None:
            @pl.when(n == 0)
            def _():
                mn_ref[0] = jnp.zeros((1, O), jnp.float32)
            mn_ref[0] = mn_ref[0] + jnp.sum(r, axis=0, keepdims=True)
            @pl.when(n == nt - 1)
            def _():
                mn_ref[0] = mn_ref[0] * jnp.float32(1.0 / N)

    outs = pl.pallas_call(
        body, grid=(B, nt), in_specs=in_specs, out_specs=out_specs,
        out_shape=out_shape)(*args)
    return list(outs)


def _knn_idx(feat, featT):
    B, N, C = feat.shape
    nt = N // TN

    def body(q_ref, all_ref, t_ref, im_ref):
        b = pl.program_id(0)
        xq = q_ref[0]
        xa = all_ref[0]
        ft = t_ref[0]
        g = lax.dot_general(xq.astype(jnp.bfloat16), xa.astype(jnp.bfloat16),
                            (((1,), (1,)), ((), ())),
                            preferred_element_type=jnp.float32)
        inner = jnp.float32(-2.0) * g
        xxq = jnp.sum(xq * xq, axis=1, keepdims=True)
        xxa = jnp.sum(ft * ft, axis=0, keepdims=True)
        d = (-xxa) - inner - xxq
        iota = lax.broadcasted_iota(jnp.int32, (TN, N), 1)
        neg = jnp.float32(-jnp.inf)
        cols = []
        for _ in range(KNN):
            m = jnp.max(d, axis=1, keepdims=True)
            c = jnp.min(jnp.where(d == m, iota, N), axis=1, keepdims=True)
            d = jnp.where(iota == c, neg, d)
            cols.append(c)
        im_ref[0] = jnp.concatenate(cols, axis=1) + b * N

    return pl.pallas_call(
        body, grid=(B, nt),
        in_specs=[pl.BlockSpec((1, TN, C), lambda b, n: (b, n, 0)),
                  pl.BlockSpec((1, N, C), lambda b, n: (b, 0, 0)),
                  pl.BlockSpec((1, C, N), lambda b, n: (b, 0, 0))],
        out_specs=pl.BlockSpec((1, TN, KNN), lambda b, n: (b, n, 0)),
        out_shape=jax.ShapeDtypeStruct((B, N, KNN), jnp.int32))(
            feat, feat, featT)


def _edge2(g, q, w6s, b2):
    B, N, O = q.shape
    GW = g.shape[2]
    nt = N // TN

    def body(g_ref, q_ref, w_ref, b_ref, y_ref, xm_ref):
        qt = q_ref[0]
        accx = jnp.full((TN, O), -jnp.inf, jnp.float32)
        accy = jnp.full((TN, O), -jnp.inf, jnp.float32)
        for k in range(KNN):
            e = _lrelu(g_ref[k][:, :O] + qt)
            accx = jnp.maximum(accx, e)
            yk = lax.dot_general(e, w_ref[...], (((1,), (1,)), ((), ())),
                                 preferred_element_type=jnp.float32)
            accy = jnp.maximum(accy, _lrelu(yk + b_ref[...]))
        y_ref[0] = accy
        xm_ref[0] = accx

    return pl.pallas_call(
        body, grid=(B, nt),
        in_specs=[
            pl.BlockSpec((KNN, TN, GW), lambda b, n: (0, b * (N // TN) + n, 0)),
            pl.BlockSpec((1, TN, O), lambda b, n: (b, n, 0)),
            pl.BlockSpec((O, O), lambda b, n: (0, 0)),
            pl.BlockSpec((1, O), lambda b, n: (0, 0))],
        out_specs=[pl.BlockSpec((1, TN, O), lambda b, n: (b, n, 0)),
                   pl.BlockSpec((1, TN, O), lambda b, n: (b, n, 0))],
        out_shape=[jax.ShapeDtypeStruct((B, N, O), jnp.float32),
                   jax.ShapeDtypeStruct((B, N, O), jnp.float32)])(
            g, q, w6s, b2.reshape(1, O))


def _head(xm, xa, y4, l1a, l1b, b6, l2, l2b, sb7, l3p, b3p, w8a):
    B = xm.shape[0]

    def body(xm_ref, xa_ref, y4_ref, l1a_ref, l1b_ref, b6_ref, l2_ref,
             l2b_ref, sb7_ref, l3_ref, b3_ref, w8_ref, xo_ref, r8_ref):
        dg = lambda a, w: lax.dot_general(
            a, w, (((1,), (1,)), ((), ())), preferred_element_type=jnp.float32)
        h = _lrelu(dg(xm_ref[...], l1a_ref[...]) + dg(xa_ref[...], l1b_ref[...])
                   + b6_ref[...])
        sb = sb7_ref[...]
        h2 = _lrelu((dg(h, l2_ref[...]) + l2b_ref[...]) * sb[0:1, :]
                    + sb[1:2, :])
        xo_ref[...] = dg(h2, l3_ref[...]) + b3_ref[...]
        r8_ref[...] = dg(y4_ref[...], w8_ref[...])

    return pl.pallas_call(
        body,
        out_shape=[jax.ShapeDtypeStruct((B, 8), jnp.float32),
                   jax.ShapeDtypeStruct((B, 512), jnp.float32)])(
            xm, xa, y4, l1a, l1b, b6.reshape(1, -1), l2, l2b.reshape(1, -1),
            sb7, l3p, b3p.reshape(1, -1), w8a)


_SC_MESH = dict(core_axis_name="c", subcore_axis_name="s")


def _sc_gather_rows(table, idx):
    R, D = table.shape
    M = idx.shape[0]
    NW = 32
    CH = 128
    per_w = M // NW
    nch = per_w // CH
    mesh = plsc.VectorSubcoreMesh(**_SC_MESH)

    @functools.partial(
        pl.kernel, mesh=mesh,
        out_type=jax.ShapeDtypeStruct((M, D), jnp.float32),
        scratch_types=[pltpu.VMEM((CH,), jnp.int32),
                       pltpu.VMEM((CH, D), jnp.float32),
                       pltpu.SemaphoreType.DMA])
    def k(table_hbm, idx_hbm, out_hbm, idx_v, rows_v, sem):
        wid = lax.axis_index("s") * 2 + lax.axis_index("c")
        wbase = wid * per_w

        def chunk(c, carry):
            base = wbase + c * CH
            pltpu.sync_copy(idx_hbm.at[pl.ds(base, CH)], idx_v)
            pltpu.async_copy(table_hbm.at[idx_v], rows_v, sem).wait()
            pltpu.sync_copy(rows_v, out_hbm.at[pl.ds(base, CH)])
            return carry

        lax.fori_loop(0, nch, chunk, jnp.int32(0))

    return k(table, idx)


def _sc_gather_max(table, idx):
    R, D = table.shape
    M = idx.shape[0]
    Q = M // KNN
    NW = 32
    CQ = 4
    ROWS = CQ * KNN
    per_w = Q // NW
    nch = per_w // CQ
    mesh = plsc.VectorSubcoreMesh(**_SC_MESH)

    @functools.partial(
        pl.kernel, mesh=mesh,
        out_type=jax.ShapeDtypeStruct((Q, D), jnp.float32),
        scratch_types=[pltpu.VMEM((ROWS,), jnp.int32),
                       pltpu.VMEM((ROWS, D), jnp.float32),
                       pltpu.VMEM((CQ, D), jnp.float32),
                       pltpu.SemaphoreType.DMA])
    def k(table_hbm, idx_hbm, out_hbm, idx_v, rows_v, max_v, sem):
        wid = lax.axis_index("s") * 2 + lax.axis_index("c")
        qbase = wid * per_w

        def chunk(c, carry):
            q0 = qbase + c * CQ
            pltpu.sync_copy(idx_hbm.at[pl.ds(q0 * KNN, ROWS)], idx_v)
            pltpu.async_copy(table_hbm.at[idx_v], rows_v, sem).wait()
            for g in range(CQ):
                for o in range(D // 16):
                    sl = pl.ds(o * 16, 16)
                    acc = rows_v[g * KNN, sl]
                    for r in range(1, KNN):
                        acc = jnp.maximum(acc, rows_v[g * KNN + r, sl])
                    max_v[g, sl] = acc
            pltpu.sync_copy(max_v, out_hbm.at[pl.ds(q0, CQ)])
            return carry

        lax.fori_loop(0, nch, chunk, jnp.int32(0))

    return k(table, idx)


def kernel(x, params):
    p = params
    B, _, N = x.shape
    R = B * N
    rs = 1.0 / jnp.sqrt(jnp.float32(1.0 + BNEPS))

    def fold(wname, gname):
        w = p[wname]
        s = p[gname] * rs
        C = w.shape[1] // 2
        return w[:, :C] * s[:, None], (w[:, C:] - w[:, :C]) * s[:, None]

    xt = jnp.transpose(x, (0, 2, 1))
    xt8 = jnp.pad(xt, ((0, 0), (0, 0), (0, 5)))

    s2 = p['bn2_g'] * rs
    w6s = p['conv6_w'] * s2[:, None]
    b2 = p['bn2_b']

    imA = _knn_idx(xt8, jnp.pad(x, ((0, 0), (0, 5), (0, 0))))
    w1A, w2A = fold('conv1_w', 'bn1_g')
    w1A = jnp.pad(w1A, ((0, 64), (0, 5)))
    w2A = jnp.pad(w2A, ((0, 0), (0, 5)))
    (PA,) = _mm(xt8, w1A)
    (QA,) = _mm(xt8, w2A, bias=p['bn1_b'])
    kmA = jnp.transpose(imA.reshape(R, KNN)).reshape(-1)
    GA = _sc_gather_rows(PA.reshape(R, 128), kmA).reshape(KNN, R, 128)
    y1, x1m = _edge2(GA, QA, w6s, b2)

    w1B, w2B = fold('conv2_w', 'bn2_g')
    w1Bp = jnp.pad(w1B, ((0, 64), (0, 0)))
    imB = _knn_idx(x1m, jnp.transpose(x1m, (0, 2, 1)))
    (PB,) = _mm(x1m, w1Bp)
    MB = _sc_gather_max(PB.reshape(R, 128), imB.reshape(-1)).reshape(B, N, 128)
    (x2,) = _mm(x1m, w2B, bias=b2, addf=MB[:, :, :64], act=True)

    imC = _knn_idx(y1, jnp.transpose(y1, (0, 2, 1)))
    (PC,) = _mm(y1, w1Bp)
    (QC,) = _mm(y1, w2B, bias=b2)
    kmC = jnp.transpose(imC.reshape(R, KNN)).reshape(-1)
    GC = _sc_gather_rows(PC.reshape(R, 128), kmC).reshape(KNN, R, 128)
    y2, _ = _edge2(GC, QC, w6s, b2)

    w1D, w2D = fold('conv3_w', 'bn3_g')
    imD = _knn_idx(x2, jnp.transpose(x2, (0, 2, 1)))
    (PD,) = _mm(x2, w1D)
    MD = _sc_gather_max(PD.reshape(R, 128), imD.reshape(-1)).reshape(B, N, 128)
    (x3,) = _mm(x2, w2D, bias=p['bn3_b'], addf=MD, act=True)

    imE = _knn_idx(y2, jnp.transpose(y2, (0, 2, 1)))
    (PE,) = _mm(y2, w1Bp)
    ME = _sc_gather_max(PE.reshape(R, 128), imE.reshape(-1)).reshape(B, N, 128)
    (y3,) = _mm(y2, w2B, bias=b2, addf=ME[:, :, :64], act=True)

    w1F, w2F = fold('conv4_w', 'bn4_g')
    imF = _knn_idx(x3, jnp.transpose(x3, (0, 2, 1)))
    (PF,) = _mm(x3, w1F)
    MF = _sc_gather_max(PF.reshape(R, 256), imF.reshape(-1)).reshape(B, N, 256)
    (x4,) = _mm(x3, w2F, bias=p['bn4_b'], addf=MF, act=True)

    s5 = p['bn5_g'] * rs
    s6 = p['bn6_g'] * rs
    s7 = p['bn7_g'] * rs
    b5, b6, b7 = p['bn5_b'], p['bn6_b'], p['bn7_b']

    xs1 = jnp.concatenate([x1m, x2, x3, x4], axis=2)
    ys1 = jnp.concatenate([y1, y2, y3], axis=2)

    (y4,) = _mm(ys1, p['conv7_w'] * s5[:, None], bias=b5, act=True,
                main_out=False, want_max=True)
    xm, xa = _mm(xs1, p['conv5_w'] * s5[:, None], bias=b5, act=True,
                 main_out=False, want_max=True, want_mean=True)
    y4, xm, xa = y4[:, 0], xm[:, 0], xa[:, 0]

    W8 = p['conv8_w'] * s6[:, None]
    l1s = p['lin1_w'] * s6[:, None]
    l3p = jnp.pad(p['lin3_w'], ((0, 3), (0, 0)))
    b3p = jnp.pad(p['lin3_b'], (0, 3))
    sb7 = jnp.stack([s7, b7])
    x_out8, row8 = _head(xm, xa, y4, l1s[:, :1024], l1s[:, 1024:],
                         b6, p['lin2_w'], p['lin2_b'], sb7, l3p, b3p,
                         W8[:, :1024])
    x_out = x_out8[:, :5]

    (yh,) = _mm(ys1, W8[:, 1024:], bias=b6, addrow=row8, act=True)
    (yh2,) = _mm(yh, p['conv9_w'] * s7[:, None], bias=b7, act=True)
    w10p = jnp.pad(p['conv10_w'], ((0, 1), (0, 0)))
    (yo,) = _mm(yh2, w10p)
    y_out = jnp.transpose(yo[:, :, :7], (0, 2, 1))
    return (x_out, y_out)


# speedup vs baseline: 11.8672x; 1.1020x over previous
import functools

import jax
import jax.numpy as jnp
from jax import lax
from jax.experimental import pallas as pl
from jax.experimental.pallas import tpu as pltpu
from jax.experimental.pallas import tpu_sc as plsc

KNN = 20
BNEPS = 1e-5
TN = 256


def _lrelu(x):
    return jnp.where(x >= 0, x, x * jnp.float32(0.2))


def _mm(a, w, bias=None, addf=None, addrow=None, act=False, main_out=True,
        want_max=False, want_mean=False):
    B, N, C = a.shape
    O = w.shape[0]
    nt = N // TN

    in_specs = [pl.BlockSpec((1, TN, C), lambda b, n: (b, n, 0)),
                pl.BlockSpec((O, C), lambda b, n: (0, 0))]
    args = [a, w]
    if bias is not None:
        in_specs.append(pl.BlockSpec((1, O), lambda b, n: (0, 0)))
        args.append(bias.reshape(1, O))
    if addf is not None:
        in_specs.append(pl.BlockSpec((1, TN, O), lambda b, n: (b, n, 0)))
        args.append(addf)
    if addrow is not None:
        in_specs.append(pl.BlockSpec((1, 1, O), lambda b, n: (b, 0, 0)))
        args.append(addrow.reshape(B, 1, O))

    out_shape = []
    out_specs = []
    if main_out:
        out_shape.append(jax.ShapeDtypeStruct((B, N, O), jnp.float32))
        out_specs.append(pl.BlockSpec((1, TN, O), lambda b, n: (b, n, 0)))
    if want_max:
        out_shape.append(jax.ShapeDtypeStruct((B, 1, O), jnp.float32))
        out_specs.append(pl.BlockSpec((1, 1, O), lambda b, n: (b, 0, 0)))
    if want_mean:
        out_shape.append(jax.ShapeDtypeStruct((B, 1, O), jnp.float32))
        out_specs.append(pl.BlockSpec((1, 1, O), lambda b, n: (b, 0, 0)))

    def body(*refs):
        refs = list(refs)
        a_ref = refs.pop(0)
        w_ref = refs.pop(0)
        b_ref = refs.pop(0) if bias is not None else None
        af_ref = refs.pop(0) if addf is not None else None
        ar_ref = refs.pop(0) if addrow is not None else None
        o_ref = refs.pop(0) if main_out else None
        mx_ref = refs.pop(0) if want_max else None
        mn_ref = refs.pop(0) if want_mean else None
        n = pl.program_id(1)
        r = lax.dot_general(a_ref[0], w_ref[...], (((1,), (1,)), ((), ())),
                            preferred_element_type=jnp.float32)
        if b_ref is not None:
            r = r + b_ref[...]
        if af_ref is not None:
            r = r + af_ref[0]
        if ar_ref is not None:
            r = r + ar_ref[0]
        if act:
            r = _lrelu(r)
        if o_ref is not None:
            o_ref[0] = r
        if mx_ref is not None:
            @pl.when(n == 0)
            def _():
                mx_ref[0] = jnp.full((1, O), -jnp.inf, jnp.float32)
            mx_ref[0] = jnp.maximum(mx_ref[0], jnp.max(r, axis=0, keepdims=True))
        if mn_ref is not None:
            @pl.when(n == 0)
            def _():
                mn_ref[0] = jnp.zeros((1, O), jnp.float32)
            mn_ref[0] = mn_ref[0] + jnp.sum(r, axis=0, keepdims=True)
            @pl.when(n == nt - 1)
            def _():
                mn_ref[0] = mn_ref[0] * jnp.float32(1.0 / N)

    outs = pl.pallas_call(
        body, grid=(B, nt), in_specs=in_specs, out_specs=out_specs,
        out_shape=out_shape)(*args)
    return list(outs)


def _knn_idx(feat, featT):
    B, N, C = feat.shape
    nt = N // TN

    def body(q_ref, all_ref, t_ref, im_ref):
        b = pl.program_id(0)
        xq = q_ref[0]
        xa = all_ref[0]
        ft = t_ref[0]
        g = lax.dot_general(xq.astype(jnp.bfloat16), xa.astype(jnp.bfloat16),
                            (((1,), (1,)), ((), ())),
                            preferred_element_type=jnp.float32)
        inner = jnp.float32(-2.0) * g
        xxq = jnp.sum(xq * xq, axis=1, keepdims=True)
        xxa = jnp.sum(ft * ft, axis=0, keepdims=True)
        d = (-xxa) - inner - xxq
        iota = lax.broadcasted_iota(jnp.int32, (TN, N), 1)
        neg = jnp.float32(-jnp.inf)
        cols = []
        for _ in range(KNN):
            m = jnp.max(d, axis=1, keepdims=True)
            c = jnp.min(jnp.where(d == m, iota, N), axis=1, keepdims=True)
            d = jnp.where(iota == c, neg, d)
            cols.append(c)
        im_ref[0] = jnp.concatenate(cols, axis=1) + b * N

    return pl.pallas_call(
        body, grid=(B, nt),
        in_specs=[pl.BlockSpec((1, TN, C), lambda b, n: (b, n, 0)),
                  pl.BlockSpec((1, N, C), lambda b, n: (b, 0, 0)),
                  pl.BlockSpec((1, C, N), lambda b, n: (b, 0, 0))],
        out_specs=pl.BlockSpec((1, TN, KNN), lambda b, n: (b, n, 0)),
        out_shape=jax.ShapeDtypeStruct((B, N, KNN), jnp.int32))(
            feat, feat, featT)


def _edge2(g, q, w6s, b2):
    B, N, O = q.shape
    GW = g.shape[2]
    nt = N // TN

    def body(g_ref, q_ref, w_ref, b_ref, y_ref, xm_ref):
        qt = q_ref[0]
        accx = jnp.full((TN, O), -jnp.inf, jnp.float32)
        accy = jnp.full((TN, O), -jnp.inf, jnp.float32)
        for k in range(KNN):
            e = _lrelu(g_ref[k][:, :O] + qt)
            accx = jnp.maximum(accx, e)
            yk = lax.dot_general(e, w_ref[...], (((1,), (1,)), ((), ())),
                                 preferred_element_type=jnp.float32)
            accy = jnp.maximum(accy, _lrelu(yk + b_ref[...]))
        y_ref[0] = accy
        xm_ref[0] = accx

    return pl.pallas_call(
        body, grid=(B, nt),
        in_specs=[
            pl.BlockSpec((KNN, TN, GW), lambda b, n: (0, b * (N // TN) + n, 0)),
            pl.BlockSpec((1, TN, O), lambda b, n: (b, n, 0)),
            pl.BlockSpec((O, O), lambda b, n: (0, 0)),
            pl.BlockSpec((1, O), lambda b, n: (0, 0))],
        out_specs=[pl.BlockSpec((1, TN, O), lambda b, n: (b, n, 0)),
                   pl.BlockSpec((1, TN, O), lambda b, n: (b, n, 0))],
        out_shape=[jax.ShapeDtypeStruct((B, N, O), jnp.float32),
                   jax.ShapeDtypeStruct((B, N, O), jnp.float32)])(
            g, q, w6s, b2.reshape(1, O))


def _head(xm, xa, y4, l1a, l1b, b6, l2, l2b, sb7, l3p, b3p, w8a):
    B = xm.shape[0]

    def body(xm_ref, xa_ref, y4_ref, l1a_ref, l1b_ref, b6_ref, l2_ref,
             l2b_ref, sb7_ref, l3_ref, b3_ref, w8_ref, xo_ref, r8_ref):
        dg = lambda a, w: lax.dot_general(
            a, w, (((1,), (1,)), ((), ())), preferred_element_type=jnp.float32)
        h = _lrelu(dg(xm_ref[...], l1a_ref[...]) + dg(xa_ref[...], l1b_ref[...])
                   + b6_ref[...])
        sb = sb7_ref[...]
        h2 = _lrelu((dg(h, l2_ref[...]) + l2b_ref[...]) * sb[0:1, :]
                    + sb[1:2, :])
        xo_ref[...] = dg(h2, l3_ref[...]) + b3_ref[...]
        r8_ref[...] = dg(y4_ref[...], w8_ref[...])

    return pl.pallas_call(
        body,
        out_shape=[jax.ShapeDtypeStruct((B, 8), jnp.float32),
                   jax.ShapeDtypeStruct((B, 512), jnp.float32)])(
            xm, xa, y4, l1a, l1b, b6.reshape(1, -1), l2, l2b.reshape(1, -1),
            sb7, l3p, b3p.reshape(1, -1), w8a)


_SC_MESH = dict(core_axis_name="c", subcore_axis_name="s")


def _sc_gather_rows(table, idx):
    R, D = table.shape
    M = idx.shape[0]
    NW = 32
    CH = 128
    NBUF = 4
    per_w = M // NW
    nch = per_w // CH
    ngrp = nch // NBUF
    mesh = plsc.VectorSubcoreMesh(**_SC_MESH)

    @functools.partial(
        pl.kernel, mesh=mesh,
        out_type=jax.ShapeDtypeStruct((M, D), jnp.float32),
        scratch_types=[pltpu.VMEM((CH,), jnp.int32)] * NBUF
                      + [pltpu.VMEM((CH, D), jnp.float32)] * NBUF
                      + [pltpu.SemaphoreType.DMA] * (3 * NBUF))
    def k(table_hbm, idx_hbm, out_hbm, i0, i1, i2, i3, b0, b1, b2, b3,
          s0, s1, s2, s3, s4, s5, s6, s7, s8, s9, s10, s11):
        ic = [i0, i1, i2, i3]
        bufs = [b0, b1, b2, b3]
        isem = [s0, s1, s2, s3]
        gsem = [s4, s5, s6, s7]
        wsem = [s8, s9, s10, s11]
        wid = lax.axis_index("s") * 2 + lax.axis_index("c")
        wbase = wid * per_w

        def ild(c, b):
            return pltpu.make_async_copy(
                idx_hbm.at[pl.ds(wbase + c * CH, CH)], ic[b], isem[b])

        def gath(b):
            return pltpu.make_async_copy(table_hbm.at[ic[b]], bufs[b], gsem[b])

        def wrt(c, b):
            return pltpu.make_async_copy(
                bufs[b], out_hbm.at[pl.ds(wbase + c * CH, CH)], wsem[b])

        for b in range(NBUF):
            ild(b, b).start()

        def grp(j, carry):
            c0 = j * NBUF
            for b in range(NBUF):
                ild(c0 + b, b).wait()
                gath(b).start()
            for b in range(NBUF):
                gath(b).wait()
                wrt(c0 + b, b).start()
            for b in range(NBUF):
                wrt(c0 + b, b).wait()

                @pl.when(c0 + NBUF + b < nch)
                def _(b=b):
                    ild(c0 + NBUF + b, b).start()
            return carry

        lax.fori_loop(0, ngrp, grp, jnp.int32(0))

    return k(table, idx)


def _sc_gather_max(table, idx):
    R, D = table.shape
    M = idx.shape[0]
    Q = M // KNN
    NW = 32
    CQ = 4
    ROWS = CQ * KNN
    NBUF = 4
    per_w = Q // NW
    nch = per_w // CQ
    ngrp = nch // NBUF
    mesh = plsc.VectorSubcoreMesh(**_SC_MESH)

    @functools.partial(
        pl.kernel, mesh=mesh,
        out_type=jax.ShapeDtypeStruct((Q, D), jnp.float32),
        scratch_types=[pltpu.VMEM((ROWS,), jnp.int32)] * NBUF
                      + [pltpu.VMEM((ROWS, D), jnp.float32)] * NBUF
                      + [pltpu.VMEM((CQ, D), jnp.float32)] * NBUF
                      + [pltpu.SemaphoreType.DMA] * (3 * NBUF))
    def k(table_hbm, idx_hbm, out_hbm, i0, i1, i2, i3, r0, r1, r2, r3,
          m0, m1, m2, m3, s0, s1, s2, s3, s4, s5, s6, s7, s8, s9, s10, s11):
        ic = [i0, i1, i2, i3]
        rows = [r0, r1, r2, r3]
        mx = [m0, m1, m2, m3]
        isem = [s0, s1, s2, s3]
        gsem = [s4, s5, s6, s7]
        wsem = [s8, s9, s10, s11]
        wid = lax.axis_index("s") * 2 + lax.axis_index("c")
        qbase = wid * per_w

        def ild(c, b):
            return pltpu.make_async_copy(
                idx_hbm.at[pl.ds((qbase + c * CQ) * KNN, ROWS)], ic[b], isem[b])

        def gath(b):
            return pltpu.make_async_copy(table_hbm.at[ic[b]], rows[b], gsem[b])

        def wrt(c, b):
            return pltpu.make_async_copy(
                mx[b], out_hbm.at[pl.ds(qbase + c * CQ, CQ)], wsem[b])

        def compute(b):
            def one_q(g, carry):
                for o in range(D // 16):
                    sl = pl.ds(o * 16, 16)
                    acc = rows[b][g * KNN, sl]
                    for r in range(1, KNN):
                        acc = jnp.maximum(acc, rows[b][g * KNN + r, sl])
                    mx[b][g, sl] = acc
                return carry
            lax.fori_loop(0, CQ, one_q, jnp.int32(0))

        for b in range(NBUF):
            ild(b, b).start()

        def grp(j, carry):
            c0 = j * NBUF
            for b in range(NBUF):
                ild(c0 + b, b).wait()
                gath(b).start()
            for b in range(NBUF):
                gath(b).wait()
                compute(b)
                wrt(c0 + b, b).start()
            for b in range(NBUF):
                wrt(c0 + b, b).wait()

                @pl.when(c0 + NBUF + b < nch)
                def _(b=b):
                    ild(c0 + NBUF + b, b).start()
            return carry

        lax.fori_loop(0, ngrp, grp, jnp.int32(0))

    return k(table, idx)


def kernel(x, params):
    p = params
    B, _, N = x.shape
    R = B * N
    rs = 1.0 / jnp.sqrt(jnp.float32(1.0 + BNEPS))

    def fold(wname, gname):
        w = p[wname]
        s = p[gname] * rs
        C = w.shape[1] // 2
        return w[:, :C] * s[:, None], (w[:, C:] - w[:, :C]) * s[:, None]

    xt = jnp.transpose(x, (0, 2, 1))
    xt8 = jnp.pad(xt, ((0, 0), (0, 0), (0, 5)))

    s2 = p['bn2_g'] * rs
    w6s = p['conv6_w'] * s2[:, None]
    b2 = p['bn2_b']

    imA = _knn_idx(xt8, jnp.pad(x, ((0, 0), (0, 5), (0, 0))))
    w1A, w2A = fold('conv1_w', 'bn1_g')
    w1A = jnp.pad(w1A, ((0, 64), (0, 5)))
    w2A = jnp.pad(w2A, ((0, 0), (0, 5)))
    (PA,) = _mm(xt8, w1A)
    (QA,) = _mm(xt8, w2A, bias=p['bn1_b'])
    kmA = jnp.transpose(imA.reshape(R, KNN)).reshape(-1)
    GA = _sc_gather_rows(PA.reshape(R, 128), kmA).reshape(KNN, R, 128)
    y1, x1m = _edge2(GA, QA, w6s, b2)

    w1B, w2B = fold('conv2_w', 'bn2_g')
    w1Bp = jnp.pad(w1B, ((0, 64), (0, 0)))
    imB = _knn_idx(x1m, jnp.transpose(x1m, (0, 2, 1)))
    (PB,) = _mm(x1m, w1Bp)
    MB = _sc_gather_max(PB.reshape(R, 128), imB.reshape(-1)).reshape(B, N, 128)
    (x2,) = _mm(x1m, w2B, bias=b2, addf=MB[:, :, :64], act=True)

    imC = _knn_idx(y1, jnp.transpose(y1, (0, 2, 1)))
    (PC,) = _mm(y1, w1Bp)
    (QC,) = _mm(y1, w2B, bias=b2)
    kmC = jnp.transpose(imC.reshape(R, KNN)).reshape(-1)
    GC = _sc_gather_rows(PC.reshape(R, 128), kmC).reshape(KNN, R, 128)
    y2, _ = _edge2(GC, QC, w6s, b2)

    w1D, w2D = fold('conv3_w', 'bn3_g')
    imD = _knn_idx(x2, jnp.transpose(x2, (0, 2, 1)))
    (PD,) = _mm(x2, w1D)
    MD = _sc_gather_max(PD.reshape(R, 128), imD.reshape(-1)).reshape(B, N, 128)
    (x3,) = _mm(x2, w2D, bias=p['bn3_b'], addf=MD, act=True)

    imE = _knn_idx(y2, jnp.transpose(y2, (0, 2, 1)))
    (PE,) = _mm(y2, w1Bp)
    ME = _sc_gather_max(PE.reshape(R, 128), imE.reshape(-1)).reshape(B, N, 128)
    (y3,) = _mm(y2, w2B, bias=b2, addf=ME[:, :, :64], act=True)

    w1F, w2F = fold('conv4_w', 'bn4_g')
    imF = _knn_idx(x3, jnp.transpose(x3, (0, 2, 1)))
    (PF,) = _mm(x3, w1F)
    MF = _sc_gather_max(PF.reshape(R, 256), imF.reshape(-1)).reshape(B, N, 256)
    (x4,) = _mm(x3, w2F, bias=p['bn4_b'], addf=MF, act=True)

    s5 = p['bn5_g'] * rs
    s6 = p['bn6_g'] * rs
    s7 = p['bn7_g'] * rs
    b5, b6, b7 = p['bn5_b'], p['bn6_b'], p['bn7_b']

    xs1 = jnp.concatenate([x1m, x2, x3, x4], axis=2)
    ys1 = jnp.concatenate([y1, y2, y3], axis=2)

    (y4,) = _mm(ys1, p['conv7_w'] * s5[:, None], bias=b5, act=True,
                main_out=False, want_max=True)
    xm, xa = _mm(xs1, p['conv5_w'] * s5[:, None], bias=b5, act=True,
                 main_out=False, want_max=True, want_mean=True)
    y4, xm, xa = y4[:, 0], xm[:, 0], xa[:, 0]

    W8 = p['conv8_w'] * s6[:, None]
    l1s = p['lin1_w'] * s6[:, None]
    l3p = jnp.pad(p['lin3_w'], ((0, 3), (0, 0)))
    b3p = jnp.pad(p['lin3_b'], (0, 3))
    sb7 = jnp.stack([s7, b7])
    x_out8, row8 = _head(xm, xa, y4, l1s[:, :1024], l1s[:, 1024:],
                         b6, p['lin2_w'], p['lin2_b'], sb7, l3p, b3p,
                         W8[:, :1024])
    x_out = x_out8[:, :5]

    (yh,) = _mm(ys1, W8[:, 1024:], bias=b6, addrow=row8, act=True)
    (yh2,) = _mm(yh, p['conv9_w'] * s7[:, None], bias=b7, act=True)
    w10p = jnp.pad(p['conv10_w'], ((0, 1), (0, 0)))
    (yo,) = _mm(yh2, w10p)
    y_out = jnp.transpose(yo[:, :, :7], (0, 2, 1))
    return (x_out, y_out)
